# Initial kernel scaffold; baseline (speedup 1.0000x reference)
import functools

import jax
import jax.numpy as jnp
from jax import lax
from jax.experimental import pallas as pl
from jax.experimental.pallas import tpu as pltpu
from jax.experimental.pallas import tpu_sc as plsc

N = 10000
E = 320000
D_IN = 128
D_ENC = 200
H = 128
L = 3

NC = 2
NS = 16
NW = NC * NS
EPT = E // NW
CH = 80
NCH = EPT // CH
ROWS_PT = N // NS

_f32 = jnp.float32


def _elu(x):
    return jnp.where(x > 0, x, jnp.expm1(jnp.minimum(x, 0.0)))


def _enc_body(x_ref, we_ref, be_ref, w0_ref, a0_ref, hs_ref, a2_ref, gm_ref):
    h = jnp.tanh(jnp.dot(x_ref[...], we_ref[...],
                         preferred_element_type=_f32) + be_ref[...])
    hs = jnp.dot(h, w0_ref[...], preferred_element_type=_f32)
    hs_ref[...] = hs
    a2 = jnp.dot(hs, a0_ref[...], preferred_element_type=_f32)
    a2_ref[...] = a2
    gm_ref[...] = jnp.full((1, 128), jnp.max(a2[:, 0:1]), _f32)


def _layer_head(agg2, s32):
    agg = agg2[0] + agg2[1]
    s_col = lax.dot_general(s32, jnp.ones((NW, 1), _f32),
                            (((0,), (0,)), ((), ())))
    return _elu(agg / (s_col + 1e-16))


def _layer_body(agg2_ref, s32_ref, w_ref, a_ref, hs_ref, a2_ref, gm_ref):
    h = _layer_head(agg2_ref[...], s32_ref[...])
    hs = jnp.dot(h, w_ref[...], preferred_element_type=_f32)
    hs_ref[...] = hs
    a2 = jnp.dot(hs, a_ref[...], preferred_element_type=_f32)
    a2_ref[...] = a2
    gm_ref[...] = jnp.full((1, 128), jnp.max(a2[:, 0:1]), _f32)


def _pool_body(agg2_ref, s32_ref, wg_ref, bg_ref, vg_ref, out_ref):
    h = _layer_head(agg2_ref[...], s32_ref[...])
    t = jnp.tanh(jnp.dot(h, wg_ref[...],
                         preferred_element_type=_f32) + bg_ref[...])
    g = jnp.dot(t, vg_ref[...], preferred_element_type=_f32)
    m = jnp.max(g)
    wexp = jnp.exp(g - m)
    z = jnp.sum(wexp)
    pooled = lax.dot_general(wexp, h, (((0,), (0,)), ((), ())))
    out_ref[...] = pooled / z


_tc_enc = pl.pallas_call(
    _enc_body,
    out_shape=(
        jax.ShapeDtypeStruct((N, H), _f32),
        jax.ShapeDtypeStruct((N, 2), _f32),
        jax.ShapeDtypeStruct((1, 128), _f32),
    ),
)

_tc_layer = pl.pallas_call(
    _layer_body,
    out_shape=(
        jax.ShapeDtypeStruct((N, H), _f32),
        jax.ShapeDtypeStruct((N, 2), _f32),
        jax.ShapeDtypeStruct((1, 128), _f32),
    ),
)

_tc_pool = pl.pallas_call(
    _pool_body,
    out_shape=jax.ShapeDtypeStruct((1, H), _f32),
)


_mesh = plsc.VectorSubcoreMesh(core_axis_name="c", subcore_axis_name="s")


@functools.partial(
    pl.kernel,
    out_type=(
        jax.ShapeDtypeStruct((NC, N, H), _f32),
        jax.ShapeDtypeStruct((NW, N), _f32),
    ),
    mesh=_mesh,
    scratch_types=[
        pltpu.VMEM((N, 2), _f32),
        pltpu.VMEM((N,), _f32),
        pltpu.VMEM((CH,), jnp.int32),
        pltpu.VMEM((CH,), jnp.int32),
        pltpu.VMEM((CH,), _f32),
        pltpu.VMEM((CH, H), _f32),
        pltpu.VMEM((128,), _f32),
        pltpu.VMEM_SHARED((N, H), _f32),
        pltpu.SemaphoreType.DMA,
    ],
)
def _sc_edges(hs_hbm, a2_hbm, gm_hbm, ei_hbm, z_hbm, agg_out, s_out,
              a2_v, s_v, srow, drow, p_row, row_buf, gm_v, agg_sh, sem):
    cid = lax.axis_index("c")
    sid = lax.axis_index("s")
    wid = cid * NS + sid

    pltpu.sync_copy(a2_hbm, a2_v)
    pltpu.sync_copy(gm_hbm.at[0], gm_v)
    pltpu.sync_copy(z_hbm.at[pl.ds(sid * ROWS_PT, ROWS_PT)],
                    agg_sh.at[pl.ds(sid * ROWS_PT, ROWS_PT)])

    zf = jnp.zeros((16,), _f32)

    def zbody(i, carry):
        s_v[pl.ds(i * 16, 16)] = zf
        return carry

    lax.fori_loop(0, N // 16, zbody, 0)
    plsc.subcore_barrier()

    gmv = gm_v[pl.ds(0, 16)]
    zi = jnp.zeros((16,), jnp.int32)
    oi = jnp.ones((16,), jnp.int32)
    ebase = wid * EPT

    def chunk(c, carry):
        base = ebase + c * CH
        pltpu.sync_copy(ei_hbm.at[0, pl.ds(base, CH)], srow)
        pltpu.sync_copy(ei_hbm.at[1, pl.ds(base, CH)], drow)
        pltpu.async_copy(hs_hbm.at[srow], row_buf, sem).wait()
        for i in range(CH // 16):
            sv = srow[pl.ds(i * 16, 16)]
            dv = drow[pl.ds(i * 16, 16)]
            asv = plsc.load_gather(a2_v, [sv, zi])
            adv = plsc.load_gather(a2_v, [dv, oi])
            t = asv + adv
            e = jnp.where(t >= 0, t, 0.2 * t)
            u = gmv + adv
            m = jnp.where(u >= 0, u, 0.2 * u)
            p = jnp.exp(e - m)
            p_row[pl.ds(i * 16, 16)] = p
            plsc.addupdate_scatter(s_v, [dv], p)

        def mul(j, c2):
            pj = p_row[j]
            for k in range(H // 16):
                row_buf[j, pl.ds(k * 16, 16)] = (
                    row_buf[j, pl.ds(k * 16, 16)] * pj)
            return c2

        lax.fori_loop(0, CH, mul, 0)
        pltpu.sync_copy(row_buf, agg_sh.at[drow], add=True)
        return carry

    lax.fori_loop(0, NCH, chunk, 0)

    pltpu.sync_copy(s_v, s_out.at[wid])
    plsc.subcore_barrier()
    pltpu.sync_copy(agg_sh.at[pl.ds(sid * ROWS_PT, ROWS_PT)],
                    agg_out.at[cid, pl.ds(sid * ROWS_PT, ROWS_PT)])


def kernel(x, edge_index, params):
    ei = edge_index.astype(jnp.int32)
    z = jnp.zeros((N, H), _f32)

    a_stack = [jnp.stack([params[f'a_src{l}'], params[f'a_dst{l}']], axis=1)
               for l in range(L)]

    hs, a2, gm = _tc_enc(x, params['W_enc'], params['b_enc'].reshape(1, -1),
                         params['W0'], a_stack[0])
    for l in range(L):
        agg2, s32 = _sc_edges(hs, a2, gm, ei, z)
        if l < L - 1:
            hs, a2, gm = _tc_layer(agg2, s32, params[f'W{l + 1}'],
                                   a_stack[l + 1])
        else:
            pooled = _tc_pool(agg2, s32, params['W_gate'],
                              params['b_gate'].reshape(1, -1),
                              params['v_gate'].reshape(-1, 1))
    return pooled.reshape(H)


# speedup vs baseline: 15.7616x; 15.7616x over previous
import functools

import jax
import jax.numpy as jnp
from jax import lax
from jax.experimental import pallas as pl
from jax.experimental.pallas import tpu as pltpu
from jax.experimental.pallas import tpu_sc as plsc

N = 10000
E = 320000
D_IN = 128
D_ENC = 200
H = 128
L = 3

NC = 2
NS = 16
NW = NC * NS
EPT = E // NW
CH = 80
NCH = EPT // CH
ROWS_A = 624
TAIL = N - NS * ROWS_A

_f32 = jnp.float32


def _elu(x):
    return jnp.where(x > 0, x, jnp.exp(jnp.minimum(x, 0.0)) - 1.0)


def _dot16(a, b):
    return jnp.dot(a.astype(jnp.bfloat16), b.astype(jnp.bfloat16),
                   preferred_element_type=_f32)


def _enc_body(x_ref, we_ref, be_ref, w0_ref, a0_ref, hs_ref, a2_ref, gm_ref):
    h = jnp.tanh(_dot16(x_ref[...], we_ref[...]) + be_ref[...])
    hs = _dot16(h, w0_ref[...])
    hs_ref[...] = hs
    a2 = _dot16(hs, a0_ref[...])
    a2_ref[...] = a2
    gm_ref[...] = jnp.full((1, 128), jnp.max(a2[:, 0:1]), _f32)


def _layer_head(agg2, s32):
    agg = agg2[0] + agg2[1]
    s_col = lax.dot_general(s32, jnp.ones((NW, 1), _f32),
                            (((0,), (0,)), ((), ())),
                            precision=lax.Precision.HIGHEST)
    return _elu(agg / (s_col + 1e-16))


def _layer_body(agg2_ref, s32_ref, w_ref, a_ref, hs_ref, a2_ref, gm_ref):
    h = _layer_head(agg2_ref[...], s32_ref[...])
    hs = _dot16(h, w_ref[...])
    hs_ref[...] = hs
    a2 = _dot16(hs, a_ref[...])
    a2_ref[...] = a2
    gm_ref[...] = jnp.full((1, 128), jnp.max(a2[:, 0:1]), _f32)


def _pool_body(agg2_ref, s32_ref, wg_ref, bg_ref, vg_ref, out_ref):
    h = _layer_head(agg2_ref[...], s32_ref[...])
    t = jnp.tanh(_dot16(h, wg_ref[...]) + bg_ref[...])
    g = _dot16(t, vg_ref[...])
    m = jnp.max(g)
    wexp = jnp.exp(g - m)
    z = jnp.sum(wexp)
    pooled = lax.dot_general(wexp, h, (((0,), (0,)), ((), ())),
                             precision=lax.Precision.HIGHEST)
    out_ref[...] = pooled / z


_tc_enc = pl.pallas_call(
    _enc_body,
    out_shape=(
        jax.ShapeDtypeStruct((N, H), _f32),
        jax.ShapeDtypeStruct((N, 2), _f32),
        jax.ShapeDtypeStruct((1, 128), _f32),
    ),
)

_tc_layer = pl.pallas_call(
    _layer_body,
    out_shape=(
        jax.ShapeDtypeStruct((N, H), _f32),
        jax.ShapeDtypeStruct((N, 2), _f32),
        jax.ShapeDtypeStruct((1, 128), _f32),
    ),
)

_tc_pool = pl.pallas_call(
    _pool_body,
    out_shape=jax.ShapeDtypeStruct((1, H), _f32),
)


_mesh = plsc.VectorSubcoreMesh(core_axis_name="c", subcore_axis_name="s")


@functools.partial(
    pl.kernel,
    out_type=(
        jax.ShapeDtypeStruct((NC, N, H), _f32),
        jax.ShapeDtypeStruct((NW * N,), _f32),
    ),
    mesh=_mesh,
    compiler_params=pltpu.CompilerParams(needs_layout_passes=False),
    scratch_types=[
        pltpu.VMEM((N,), _f32),
        pltpu.VMEM((N,), _f32),
        pltpu.VMEM((N,), _f32),
        pltpu.VMEM((CH,), jnp.int32),
        pltpu.VMEM((CH,), jnp.int32),
        pltpu.VMEM((CH, H), _f32),
        pltpu.VMEM((128,), _f32),
        pltpu.VMEM_SHARED((N, H), _f32),
        pltpu.SemaphoreType.DMA,
    ],
)
def _sc_edges(hs_hbm, asrc_hbm, adst_hbm, gm_hbm, esrc_hbm, edst_hbm, z_hbm,
              agg_out, s_out,
              asrc_v, adst_v, s_v, srow, drow, row_buf, gm_v, agg_sh,
              sem):
    cid = lax.axis_index("c")
    sid = lax.axis_index("s")
    wid = cid * NS + sid

    pltpu.sync_copy(asrc_hbm, asrc_v)
    pltpu.sync_copy(adst_hbm, adst_v)
    pltpu.sync_copy(gm_hbm.at[0], gm_v)
    pltpu.sync_copy(z_hbm.at[pl.ds(sid * ROWS_A, ROWS_A)],
                    agg_sh.at[pl.ds(sid * ROWS_A, ROWS_A)])

    @pl.when(sid == NS - 1)
    def _():
        pltpu.sync_copy(z_hbm.at[pl.ds(NS * ROWS_A, TAIL)],
                        agg_sh.at[pl.ds(NS * ROWS_A, TAIL)])

    zf = jnp.zeros((16,), _f32)

    def zbody(i, carry):
        s_v[pl.ds(i * 16, 16)] = zf
        return carry

    lax.fori_loop(0, N // 16, zbody, 0)
    plsc.subcore_barrier()

    gmv = gm_v[pl.ds(0, 16)]
    ebase = wid * EPT

    def chunk(c, carry):
        base = ebase + c * CH
        pltpu.sync_copy(esrc_hbm.at[pl.ds(base, CH)], srow)
        pltpu.sync_copy(edst_hbm.at[pl.ds(base, CH)], drow)
        pltpu.async_copy(hs_hbm.at[srow], row_buf, sem).wait()
        for i in range(CH // 16):
            sv = srow[pl.ds(i * 16, 16)]
            dv = drow[pl.ds(i * 16, 16)]
            asv = plsc.load_gather(asrc_v, [sv])
            adv = plsc.load_gather(adst_v, [dv])
            t = asv + adv
            e = jnp.where(t >= 0, t, 0.2 * t)
            u = gmv + adv
            m = jnp.where(u >= 0, u, 0.2 * u)
            p = jnp.exp(e - m)
            plsc.addupdate_scatter(s_v, [dv], p)
            for j in range(16):
                pj = jnp.take_along_axis(p, jnp.full((16,), j, jnp.int32), 0)
                r = i * 16 + j
                for k in range(H // 16):
                    row_buf[r, pl.ds(k * 16, 16)] = (
                        row_buf[r, pl.ds(k * 16, 16)] * pj)

        pltpu.sync_copy(row_buf, agg_sh.at[drow], add=True)
        return carry

    lax.fori_loop(0, NCH, chunk, 0)

    pltpu.sync_copy(s_v, s_out.at[pl.ds(wid * N, N)])
    plsc.subcore_barrier()
    pltpu.sync_copy(agg_sh.at[pl.ds(sid * ROWS_A, ROWS_A)],
                    agg_out.at[cid, pl.ds(sid * ROWS_A, ROWS_A)])

    @pl.when(sid == NS - 1)
    def _():
        pltpu.sync_copy(agg_sh.at[pl.ds(NS * ROWS_A, TAIL)],
                        agg_out.at[cid, pl.ds(NS * ROWS_A, TAIL)])


def kernel(x, edge_index, params):
    ei = edge_index.astype(jnp.int32)
    esrc = ei[0]
    edst = ei[1]
    z = jnp.zeros((N, H), _f32)

    a_stack = [jnp.stack([params[f'a_src{l}'], params[f'a_dst{l}']], axis=1)
               for l in range(L)]

    hs, a2, gm = _tc_enc(x, params['W_enc'], params['b_enc'].reshape(1, -1),
                         params['W0'], a_stack[0])
    for l in range(L):
        agg2, s_flat = _sc_edges(hs, a2[:, 0], a2[:, 1], gm, esrc, edst, z)
        s32 = s_flat.reshape(NW, N)
        if l < L - 1:
            hs, a2, gm = _tc_layer(agg2, s32, params[f'W{l + 1}'],
                                   a_stack[l + 1])
        else:
            pooled = _tc_pool(agg2, s32, params['W_gate'],
                              params['b_gate'].reshape(1, -1),
                              params['v_gate'].reshape(-1, 1))
    return pooled.reshape(H)


# speedup vs baseline: 19.4158x; 1.2318x over previous
import functools

import jax
import jax.numpy as jnp
from jax import lax
from jax.experimental import pallas as pl
from jax.experimental.pallas import tpu as pltpu
from jax.experimental.pallas import tpu_sc as plsc

N = 10000
E = 320000
D_IN = 128
D_ENC = 200
H = 128
L = 3

NC = 2
NS = 16
NW = NC * NS
EPT = E // NW
CH = 80
NCH = EPT // CH
ROWS_A = 624
TAIL = N - NS * ROWS_A

_f32 = jnp.float32


def _elu(x):
    return jnp.where(x > 0, x, jnp.exp(jnp.minimum(x, 0.0)) - 1.0)


def _dot16(a, b):
    return jnp.dot(a.astype(jnp.bfloat16), b.astype(jnp.bfloat16),
                   preferred_element_type=_f32)


def _enc_body(x_ref, we_ref, be_ref, w0_ref, a0_ref, hs_ref, a2_ref, gm_ref):
    h = jnp.tanh(_dot16(x_ref[...], we_ref[...]) + be_ref[...])
    hs = _dot16(h, w0_ref[...])
    hs_ref[...] = hs
    a2 = _dot16(hs, a0_ref[...])
    a2_ref[...] = a2
    gm_ref[...] = jnp.full((1, 128), jnp.max(a2[:, 0:1]), _f32)


def _layer_head(agg2, s32):
    agg = agg2[0] + agg2[1]
    s_col = lax.dot_general(s32, jnp.ones((NW, 1), _f32),
                            (((0,), (0,)), ((), ())),
                            precision=lax.Precision.HIGHEST)
    return _elu(agg / (s_col + 1e-16))


def _layer_body(agg2_ref, s32_ref, w_ref, a_ref, hs_ref, a2_ref, gm_ref):
    h = _layer_head(agg2_ref[...], s32_ref[...])
    hs = _dot16(h, w_ref[...])
    hs_ref[...] = hs
    a2 = _dot16(hs, a_ref[...])
    a2_ref[...] = a2
    gm_ref[...] = jnp.full((1, 128), jnp.max(a2[:, 0:1]), _f32)


def _pool_body(agg2_ref, s32_ref, wg_ref, bg_ref, vg_ref, out_ref):
    h = _layer_head(agg2_ref[...], s32_ref[...])
    t = jnp.tanh(_dot16(h, wg_ref[...]) + bg_ref[...])
    g = _dot16(t, vg_ref[...])
    m = jnp.max(g)
    wexp = jnp.exp(g - m)
    z = jnp.sum(wexp)
    pooled = lax.dot_general(wexp, h, (((0,), (0,)), ((), ())),
                             precision=lax.Precision.HIGHEST)
    out_ref[...] = pooled / z


_tc_enc = pl.pallas_call(
    _enc_body,
    out_shape=(
        jax.ShapeDtypeStruct((N, H), _f32),
        jax.ShapeDtypeStruct((N, 2), _f32),
        jax.ShapeDtypeStruct((1, 128), _f32),
    ),
)

_tc_layer = pl.pallas_call(
    _layer_body,
    out_shape=(
        jax.ShapeDtypeStruct((N, H), _f32),
        jax.ShapeDtypeStruct((N, 2), _f32),
        jax.ShapeDtypeStruct((1, 128), _f32),
    ),
)

_tc_pool = pl.pallas_call(
    _pool_body,
    out_shape=jax.ShapeDtypeStruct((1, H), _f32),
)


_mesh = plsc.VectorSubcoreMesh(core_axis_name="c", subcore_axis_name="s")


@functools.partial(
    pl.kernel,
    out_type=(
        jax.ShapeDtypeStruct((NC, N, H), _f32),
        jax.ShapeDtypeStruct((NW * N,), _f32),
    ),
    mesh=_mesh,
    compiler_params=pltpu.CompilerParams(needs_layout_passes=False,
                                     internal_scratch_in_bytes=0),
    scratch_types=[
        pltpu.VMEM((3 * N,), _f32),
        pltpu.VMEM((4 * CH,), jnp.int32),
        pltpu.VMEM((2 * CH, H), _f32),
        pltpu.VMEM_SHARED((N, H), _f32),
        pltpu.SemaphoreType.DMA,
        pltpu.SemaphoreType.DMA,
        pltpu.SemaphoreType.DMA,
        pltpu.SemaphoreType.DMA,
    ],
)
def _sc_edges(hs_hbm, asrc_hbm, adst_hbm, esrc_hbm, edst_hbm, z_hbm,
              agg_out, s_out,
              av, idx, rowbuf,
              agg_sh, sg0, sg1, ss0, ss1):
    cid = lax.axis_index("c")
    sid = lax.axis_index("s")
    wid = cid * NS + sid
    ebase = wid * EPT

    pltpu.sync_copy(asrc_hbm, av.at[pl.ds(0, N)])
    pltpu.sync_copy(adst_hbm, av.at[pl.ds(N, N)])
    pltpu.sync_copy(z_hbm.at[pl.ds(sid * ROWS_A, ROWS_A)],
                    agg_sh.at[pl.ds(sid * ROWS_A, ROWS_A)])

    @pl.when(sid == NS - 1)
    def _():
        pltpu.sync_copy(z_hbm.at[pl.ds(NS * ROWS_A, TAIL)],
                        agg_sh.at[pl.ds(NS * ROWS_A, TAIL)])

    zf = jnp.zeros((16,), _f32)
    noff = jnp.full((16,), N, jnp.int32)
    soff = jnp.full((16,), 2 * N, jnp.int32)

    def zbody(i, carry):
        av[pl.ds(2 * N + i * 16, 16)] = zf
        return carry

    lax.fori_loop(0, N // 16, zbody, 0)
    plsc.subcore_barrier()

    sgs = (sg0, sg1)
    sss = (ss0, ss1)

    def srow(b):
        return idx.at[pl.ds(2 * b * CH, CH)]

    def drow(b):
        return idx.at[pl.ds((2 * b + 1) * CH, CH)]

    def buf(b):
        return rowbuf.at[pl.ds(b * CH, CH)]

    def copy_idx(c, b):
        base = ebase + c * CH
        pltpu.sync_copy(esrc_hbm.at[pl.ds(base, CH)], srow(b))
        pltpu.sync_copy(edst_hbm.at[pl.ds(base, CH)], drow(b))

    def start_gather(b):
        pltpu.async_copy(hs_hbm.at[srow(b)], buf(b), sgs[b])

    def wait_gather(b):
        pltpu.make_async_copy(hs_hbm.at[srow(b)], buf(b), sgs[b]).wait()

    def start_scatter(b):
        pltpu.async_copy(buf(b), agg_sh.at[drow(b)], sss[b], add=True)

    def wait_scatter(b):
        pltpu.make_async_copy(buf(b), agg_sh.at[drow(b)], sss[b]).wait()

    def process(b):
        for i in range(CH // 16):
            sv = idx[pl.ds(2 * b * CH + i * 16, 16)]
            dv = idx[pl.ds((2 * b + 1) * CH + i * 16, 16)]
            asv = plsc.load_gather(av, [sv])
            adv = plsc.load_gather(av, [dv + noff])
            t = asv + adv
            e = jnp.where(t >= 0, t, 0.2 * t)
            m = jnp.where(adv >= 0, adv, 0.2 * adv)
            p = jnp.exp(e - m)
            plsc.addupdate_scatter(av, [dv + soff], p)
            for j in range(16):
                pj = jnp.take_along_axis(p, jnp.full((16,), j, jnp.int32), 0)
                r = b * CH + i * 16 + j
                for k in range(H // 16):
                    rowbuf[r, pl.ds(k * 16, 16)] = (
                        rowbuf[r, pl.ds(k * 16, 16)] * pj)

    copy_idx(0, 0)
    start_gather(0)

    def step(c, b, guarded_wait):
        nb = 1 - b
        if guarded_wait is None:
            wait_scatter(nb)
        else:
            @pl.when(guarded_wait)
            def _():
                wait_scatter(nb)
        copy_idx(c + 1, nb)
        start_gather(nb)
        wait_gather(b)
        process(b)
        start_scatter(b)

    def body(k, carry):
        step(2 * k, 0, k >= 1)
        step(2 * k + 1, 1, None)
        return carry

    lax.fori_loop(0, NCH // 2, body, 0)
    wait_scatter(1)
    wait_gather(0)
    process(0)
    start_scatter(0)
    wait_scatter(0)

    pltpu.sync_copy(av.at[pl.ds(2 * N, N)], s_out.at[pl.ds(wid * N, N)])
    plsc.subcore_barrier()
    pltpu.sync_copy(agg_sh.at[pl.ds(sid * ROWS_A, ROWS_A)],
                    agg_out.at[cid, pl.ds(sid * ROWS_A, ROWS_A)])

    @pl.when(sid == NS - 1)
    def _():
        pltpu.sync_copy(agg_sh.at[pl.ds(NS * ROWS_A, TAIL)],
                        agg_out.at[cid, pl.ds(NS * ROWS_A, TAIL)])


def kernel(x, edge_index, params):
    ei = edge_index.astype(jnp.int32)
    esrc = ei[0]
    edst = ei[1]
    z = jnp.zeros((N, H), _f32)

    a_stack = [jnp.stack([params[f'a_src{l}'], params[f'a_dst{l}']], axis=1)
               for l in range(L)]

    hs, a2, gm = _tc_enc(x, params['W_enc'], params['b_enc'].reshape(1, -1),
                         params['W0'], a_stack[0])
    for l in range(L):
        gmx = gm[0, 0]
        agg2, s_flat = _sc_edges(hs, a2[:, 0] - gmx, a2[:, 1] + gmx,
                                 esrc, edst, z)
        s32 = s_flat.reshape(NW, N)
        if l < L - 1:
            hs, a2, gm = _tc_layer(agg2, s32, params[f'W{l + 1}'],
                                   a_stack[l + 1])
        else:
            pooled = _tc_pool(agg2, s32, params['W_gate'],
                              params['b_gate'].reshape(1, -1),
                              params['v_gate'].reshape(-1, 1))
    return pooled.reshape(H)
